# baseline (speedup 1.0000x reference)
import jax
import jax.numpy as jnp
from jax import lax
from jax.experimental import pallas as pl
from jax.experimental.pallas import tpu as pltpu
from jax.experimental.pallas import tpu_sc as plsc

_B = 4096
_C = 1000
_NW = 32
_RW = _B // _NW
_NG = _RW // 16
_NBINS = 128
_SCL = 32.0
_NEG = -3.0e38


def _sc_body(x_hbm, tgt_hbm, lcn_hbm, kpc_hbm,
             om_hbm, oz_hbm, olat_hbm, os_hbm, oin_hbm,
             xbuf, lcnbuf, kpcbuf, tgtbuf, hist, ehist, ebuf,
             mbuf, zbuf, latbuf, sbuf, inbuf, dsem):
    cid = lax.axis_index("c")
    sid = lax.axis_index("s")
    wid = sid * 2 + cid
    base = wid * _RW

    pltpu.sync_copy(lcn_hbm, lcnbuf)
    pltpu.sync_copy(kpc_hbm, kpcbuf)
    pltpu.sync_copy(tgt_hbm.at[pl.ds(base, _RW)], tgtbuf)

    lane = lax.iota(jnp.int32, 16)
    lane_c = lane * jnp.int32(_C)
    zeros_i = jnp.zeros((16,), jnp.int32)
    ones_i = jnp.ones((16,), jnp.int32)
    zeros_f = jnp.zeros((16,), jnp.float32)
    gwords = 16 * _C

    pltpu.async_copy(x_hbm.at[pl.ds(base * _C, gwords)],
                     xbuf.at[pl.ds(0, gwords)], dsem)

    def suffix_scan(k16):
        csum = []
        esum = []
        for ci in range(16):
            s = hist[pl.ds(ci * 128, 16)]
            e = ehist[pl.ds(ci * 128, 16)]
            for j in range(1, 8):
                s = s + hist[pl.ds(ci * 128 + j * 16, 16)]
                e = e + ehist[pl.ds(ci * 128 + j * 16, 16)]
            csum.append(s)
            esum.append(e)
        sufs = [None] * 16
        sufe = [None] * 16
        accv = zeros_i
        acce = zeros_f
        for ci in range(15, -1, -1):
            accv = accv + csum[ci]
            acce = acce + esum[ci]
            sufs[ci] = accv
            sufe[ci] = acce
        cnt_total = accv
        e_total = acce
        found = jnp.zeros((16,), jnp.bool_)
        cch = zeros_i
        nabc = zeros_i
        eabc = zeros_f
        for ci in range(15, -1, -1):
            above_c = sufs[ci + 1] if ci < 15 else zeros_i
            above_e = sufe[ci + 1] if ci < 15 else zeros_f
            crossed = jnp.logical_and(sufs[ci] >= k16,
                                      jnp.logical_not(found))
            cch = jnp.where(crossed, jnp.int32(ci), cch)
            nabc = jnp.where(crossed, above_c, nabc)
            eabc = jnp.where(crossed, above_e, eabc)
            found = jnp.logical_or(found, crossed)
        found2 = jnp.zeros((16,), jnp.bool_)
        bstar = zeros_i
        nab = zeros_i
        eab = zeros_f
        einc = zeros_f
        accv = nabc
        acce = eabc
        for j in range(7, -1, -1):
            b16 = cch * 8 + jnp.int32(j)
            cntb = plsc.load_gather(hist, [b16 * 16 + lane])
            eb = plsc.load_gather(ehist, [b16 * 16 + lane])
            accn = accv + cntb
            ecn = acce + eb
            crossed = jnp.logical_and(accn >= k16,
                                      jnp.logical_not(found2))
            bstar = jnp.where(crossed, b16, bstar)
            nab = jnp.where(crossed, accv, nab)
            eab = jnp.where(crossed, acce, eab)
            einc = jnp.where(crossed, ecn, einc)
            found2 = jnp.logical_or(found2, crossed)
            accv = accn
            acce = ecn
        return bstar, k16 - nab, eab, einc, cnt_total, e_total

    def group_body(g, _):
        rb = pl.multiple_of(g * 16, 16)
        pbase = (g % 2) * gwords
        pltpu.make_async_copy(
            x_hbm.at[pl.ds((base + rb) * _C, gwords)],
            xbuf.at[pl.ds(pbase, gwords)], dsem).wait()

        @pl.when(g < _NG - 1)
        def _start_next():
            pltpu.async_copy(
                x_hbm.at[pl.ds((base + rb + 16) * _C, gwords)],
                xbuf.at[pl.ds(((g + 1) % 2) * gwords, gwords)], dsem)

        plane_c = lane_c + pbase
        tgt16 = tgtbuf[pl.ds(rb, 16)]
        k16 = jnp.minimum(plsc.load_gather(kpcbuf, [tgt16]), jnp.int32(_C))

        neg16 = jnp.full((16,), _NEG, jnp.float32)

        @plsc.parallel_loop(0, _C, step=8, unroll=4, carry=(neg16,) * 8)
        def pa(i, st):
            idx = plane_c + i
            acc = list(st)
            for u in range(8):
                v = plsc.load_gather(xbuf, [idx + jnp.int32(u)])
                acc[u] = jnp.maximum(acc[u], v)
            return tuple(acc)
        a = pa
        hi16 = jnp.maximum(
            jnp.maximum(jnp.maximum(a[0], a[1]), jnp.maximum(a[2], a[3])),
            jnp.maximum(jnp.maximum(a[4], a[5]), jnp.maximum(a[6], a[7])))
        base16 = jnp.float32(_NBINS) - hi16 * jnp.float32(_SCL)

        for i in range(_NBINS):
            hist[pl.ds(i * 16, 16)] = zeros_i
            ehist[pl.ds(i * 16, 16)] = zeros_f

        @plsc.parallel_loop(0, _C, step=8, unroll=4)
        def pb1(i):
            idx = plane_c + i
            eidx = lane_c + i
            cidx = jnp.full((16,), i, jnp.int32)
            for u in range(8):
                v = plsc.load_gather(xbuf, [idx + jnp.int32(u)])
                lc = plsc.load_gather(lcnbuf, [cidx + jnp.int32(u)])
                e = jnp.exp(v - hi16 + lc)
                t = v * jnp.float32(_SCL) + base16
                t = jnp.minimum(jnp.maximum(t, jnp.float32(0.0)),
                                jnp.float32(_NBINS - 1))
                hidx = t.astype(jnp.int32) * 16 + lane
                plsc.addupdate_scatter(hist, [hidx], ones_i)
                plsc.addupdate_scatter(ehist, [hidx], e)
                plsc.store_scatter(ebuf, [eidx + jnp.int32(u)], e)

        bstar16, krem16, eab16, _, _, z16 = suffix_scan(k16)

        bstar_f = bstar16.astype(jnp.float32)
        low16 = jnp.where(bstar16 == 0, _NEG, bstar_f)
        high16 = jnp.where(bstar16 == jnp.int32(_NBINS - 1), -_NEG,
                           bstar_f + jnp.float32(1.0))

        for i in range(_NBINS):
            hist[pl.ds(i * 16, 16)] = zeros_i
            ehist[pl.ds(i * 16, 16)] = zeros_f

        @plsc.parallel_loop(0, _C, step=8, unroll=4)
        def pb2(i):
            idx = plane_c + i
            eidx = lane_c + i
            for u in range(8):
                v = plsc.load_gather(xbuf, [idx + jnp.int32(u)])
                e = plsc.load_gather(ebuf, [eidx + jnp.int32(u)])
                t = v * jnp.float32(_SCL) + base16
                msk = jnp.logical_and(t >= low16, t < high16)
                t2 = (t - bstar_f) * jnp.float32(_NBINS)
                t2 = jnp.minimum(jnp.maximum(t2, jnp.float32(0.0)),
                                 jnp.float32(_NBINS - 1))
                hidx = t2.astype(jnp.int32) * 16 + lane
                plsc.addupdate_scatter(hist, [hidx], ones_i, mask=msk)
                plsc.addupdate_scatter(ehist, [hidx], e, mask=msk)

        b2star16, _, _, einc16, _, _ = suffix_scan(krem16)

        s16 = eab16 + einc16

        xt16 = plsc.load_gather(xbuf, [plane_c + tgt16])
        lcnt16 = plsc.load_gather(lcnbuf, [tgt16])
        lat16 = xt16 + lcnt16
        tt = xt16 * jnp.float32(_SCL) + base16
        ttc = jnp.minimum(jnp.maximum(tt, jnp.float32(0.0)),
                          jnp.float32(_NBINS - 1))
        bit = ttc.astype(jnp.int32)
        t2t = (tt - bstar_f) * jnp.float32(_NBINS)
        t2t = jnp.minimum(jnp.maximum(t2t, jnp.float32(0.0)),
                          jnp.float32(_NBINS - 1))
        b2t = t2t.astype(jnp.int32)
        member_t = jnp.logical_or(
            bit > bstar16,
            jnp.logical_and(bit == bstar16, b2t >= b2star16))
        in16 = jnp.where(member_t, jnp.float32(1.0), jnp.float32(0.0))

        sl = pl.ds(rb, 16)
        mbuf[sl] = hi16
        zbuf[sl] = z16
        latbuf[sl] = lat16
        sbuf[sl] = s16
        inbuf[sl] = in16
        return 0

    lax.fori_loop(0, _NG, group_body, 0)

    osl = pl.ds(base, _RW)
    pltpu.sync_copy(mbuf, om_hbm.at[osl])
    pltpu.sync_copy(zbuf, oz_hbm.at[osl])
    pltpu.sync_copy(latbuf, olat_hbm.at[osl])
    pltpu.sync_copy(sbuf, os_hbm.at[osl])
    pltpu.sync_copy(inbuf, oin_hbm.at[osl])


def kernel(logit, target, log_cls_num, k_per_class):
    f32 = jnp.float32
    i32 = jnp.int32
    mesh = plsc.VectorSubcoreMesh(core_axis_name="c", subcore_axis_name="s",
                                  num_cores=2, num_subcores=16)
    sck = pl.kernel(
        _sc_body,
        out_type=tuple(jax.ShapeDtypeStruct((_B,), f32) for _ in range(5)),
        mesh=mesh,
        scratch_types=[
            pltpu.VMEM((2 * 16 * _C,), f32),
            pltpu.VMEM((_C,), f32),
            pltpu.VMEM((_C,), i32),
            pltpu.VMEM((_RW,), i32),
            pltpu.VMEM((_NBINS * 16,), i32),
            pltpu.VMEM((_NBINS * 16,), f32),
            pltpu.VMEM((16 * _C,), f32),
            pltpu.VMEM((_RW,), f32),
            pltpu.VMEM((_RW,), f32),
            pltpu.VMEM((_RW,), f32),
            pltpu.VMEM((_RW,), f32),
            pltpu.VMEM((_RW,), f32),
            pltpu.SemaphoreType.DMA,
        ],
        compiler_params=pltpu.CompilerParams(needs_layout_passes=False),
    )
    m, z, lat, s, inn = sck(logit.reshape(_B * _C), target, log_cls_num,
                            k_per_class)
    logz = m + jnp.log(z)
    lf = logz - lat
    pt = jnp.exp(lat - logz)
    num = jnp.where(inn > 0.5, pt + f32(1e-6), f32(1e-6))
    lt = jnp.log(s / z + f32(_C * 1e-6)) - jnp.log(num)
    return jnp.mean(0.5 * (lf + lt))
